# Initial kernel scaffold; baseline (speedup 1.0000x reference)
import jax
import jax.numpy as jnp
from jax.experimental import pallas as pl


def kernel(atom_type, residue_type, residue_pos, atom_table, residue_table, pos_table, W1, b1, W2, b2):
    raise NotImplementedError("write your pallas kernel here")


# speedup vs baseline: 11.3983x; 11.3983x over previous
import functools

import jax
import jax.numpy as jnp
from jax import lax
from jax.experimental import pallas as pl
from jax.experimental.pallas import tpu as pltpu
from jax.experimental.pallas import tpu_sc as plsc

NI, NJ, NK = 54, 20, 23
C = 64
NJK = NJ * NK
NCOMBO = NI * NJK

CH = 128


def _table_body(at_ref, rt_ref, pt_ref, w1_ref, b1_ref, w2_ref, b2_ref, f_ref):
    i = pl.program_id(0)
    a_row = jnp.dot(at_ref[pl.ds(i, 1), :], w1_ref[0:C, :],
                    preferred_element_type=jnp.float32)
    r_proj = jnp.dot(rt_ref[...], w1_ref[C:2 * C, :], preferred_element_type=jnp.float32)
    p_proj = jnp.dot(pt_ref[...], w1_ref[2 * C:3 * C, :], preferred_element_type=jnp.float32)
    rj = lax.broadcasted_iota(jnp.int32, (NJK, NJ), 0)
    cj = lax.broadcasted_iota(jnp.int32, (NJK, NJ), 1)
    oh_j = (rj // NK == cj).astype(jnp.float32)
    rk = lax.broadcasted_iota(jnp.int32, (NJK, NK), 0)
    ck = lax.broadcasted_iota(jnp.int32, (NJK, NK), 1)
    oh_k = (rk % NK == ck).astype(jnp.float32)
    pre = (jnp.dot(oh_j, r_proj, preferred_element_type=jnp.float32)
           + jnp.dot(oh_k, p_proj, preferred_element_type=jnp.float32)
           + a_row + b1_ref[...])
    h = 0.5 * pre * (1.0 + lax.erf(pre * (2.0 ** -0.5)))
    out = jnp.dot(h, w2_ref[...], preferred_element_type=jnp.float32) + b2_ref[...]
    f_ref[...] = out[None]


def _build_table(atom_table, residue_table, pos_table, W1, b1, W2, b2):
    full = lambda shape: pl.BlockSpec(shape, lambda i: (0,) * len(shape))
    return pl.pallas_call(
        _table_body,
        grid=(NI,),
        in_specs=[
            full((NI, C)),
            full((NJ, C)),
            full((NK, C)),
            full((3 * C, C)),
            full((1, C)),
            full((C, C)),
            full((1, C)),
        ],
        out_specs=pl.BlockSpec((1, NJK, C), lambda i: (i, 0, 0)),
        out_shape=jax.ShapeDtypeStruct((NI, NJK, C), jnp.float32),
    )(atom_table, residue_table, pos_table, W1,
      b1.reshape(1, C), W2, b2.reshape(1, C))


def _make_gather(num_tokens):
    info = plsc.get_sparse_core_info()
    nc, ns = info.num_cores, info.num_subcores
    nw = nc * ns
    nchunk = num_tokens // (nw * CH)
    mesh = plsc.VectorSubcoreMesh(core_axis_name="c", subcore_axis_name="s")

    @functools.partial(
        pl.kernel,
        out_type=jax.ShapeDtypeStruct((num_tokens, C), jnp.float32),
        mesh=mesh,
        compiler_params=pltpu.CompilerParams(use_tc_tiling_on_sc=False),
        scratch_types=[
            pltpu.VMEM((nchunk, CH), jnp.int32),
            pltpu.VMEM((CH, C), jnp.float32),
            pltpu.SemaphoreType.DMA,
        ],
    )
    def gather_k(f_hbm, idx_hbm, out_hbm, idx_v, rows_v, sem):
        wid = lax.axis_index("s") * nc + lax.axis_index("c")
        rowbase = wid * nchunk
        pltpu.sync_copy(idx_hbm.at[pl.ds(rowbase, nchunk)], idx_v)

        def body(j, carry):
            pltpu.async_copy(f_hbm.at[idx_v.at[j]], rows_v, sem).wait()
            pltpu.sync_copy(rows_v, out_hbm.at[pl.ds((rowbase + j) * CH, CH)])
            return carry

        lax.fori_loop(0, nchunk, body, 0)

    return gather_k


def kernel(atom_type, residue_type, residue_pos, atom_table, residue_table,
           pos_table, W1, b1, W2, b2):
    B, L = atom_type.shape
    num_tokens = B * L
    F = _build_table(atom_table, residue_table, pos_table, W1, b1, W2, b2)
    F = F.reshape(NCOMBO, C)
    cidx = (atom_type * NJK + residue_type * NK + residue_pos).astype(jnp.int32)
    cidx = cidx.reshape(num_tokens // CH, CH)
    out = _make_gather(num_tokens)(F, cidx)
    return out.reshape(B, L, C)


# speedup vs baseline: 70.2164x; 6.1603x over previous
import functools

import jax
import jax.numpy as jnp
from jax import lax
from jax.experimental import pallas as pl
from jax.experimental.pallas import tpu as pltpu
from jax.experimental.pallas import tpu_sc as plsc

NI, NJ, NK = 54, 20, 23
C = 64
NJK = NJ * NK
NCOMBO = NI * NJK


def _table_body(at_ref, rt_ref, pt_ref, w1_ref, b1_ref, w2_ref, b2_ref, f_ref):
    hi = jnp.float32
    ct = lambda a, b, dims: lax.dot_general(a, b, (dims, ((), ())),
                                            preferred_element_type=hi)
    a_t = ct(w1_ref[0:C, :], at_ref[...], ((0,), (1,)))
    r_t = ct(w1_ref[C:2 * C, :], rt_ref[...], ((0,), (1,)))
    p_t = ct(w1_ref[2 * C:3 * C, :], pt_ref[...], ((0,), (1,)))
    combo_i = lax.broadcasted_iota(jnp.int32, (NI, NCOMBO), 1)
    oh_i = (combo_i // NJK == lax.broadcasted_iota(jnp.int32, (NI, NCOMBO), 0)
            ).astype(jnp.float32)
    combo_j = lax.broadcasted_iota(jnp.int32, (NJ, NCOMBO), 1)
    oh_j = ((combo_j // NK) % NJ == lax.broadcasted_iota(jnp.int32, (NJ, NCOMBO), 0)
            ).astype(jnp.float32)
    combo_k = lax.broadcasted_iota(jnp.int32, (NK, NCOMBO), 1)
    oh_k = (combo_k % NK == lax.broadcasted_iota(jnp.int32, (NK, NCOMBO), 0)
            ).astype(jnp.float32)
    pre = (ct(a_t, oh_i, ((1,), (0,))) + ct(r_t, oh_j, ((1,), (0,)))
           + ct(p_t, oh_k, ((1,), (0,))) + b1_ref[...])
    h = 0.5 * pre * (1.0 + lax.erf(pre * (2.0 ** -0.5)))
    sel_r = lax.broadcasted_iota(jnp.int32, (C, C // 2), 0)
    sel_c = lax.broadcasted_iota(jnp.int32, (C, C // 2), 1)
    s_e = (sel_r == 2 * sel_c).astype(jnp.float32)
    s_o = (sel_r == 2 * sel_c + 1).astype(jnp.float32)
    w2e = ct(w2_ref[...], s_e, ((1,), (0,)))
    w2o = ct(w2_ref[...], s_o, ((1,), (0,)))
    b2e = ct(s_e, b2_ref[...], ((0,), (0,)))
    b2o = ct(s_o, b2_ref[...], ((0,), (0,)))
    o_e = ct(w2e, h, ((0,), (0,))) + b2e
    o_o = ct(w2o, h, ((0,), (0,))) + b2o
    u_e = lax.bitcast_convert_type(o_e.astype(jnp.bfloat16), jnp.uint16
                                   ).astype(jnp.uint32)
    u_o = lax.bitcast_convert_type(o_o.astype(jnp.bfloat16), jnp.uint16
                                   ).astype(jnp.uint32)
    f_ref[...] = lax.bitcast_convert_type(u_e | (u_o << 16), jnp.int32)


def _build_table_packed(atom_table, residue_table, pos_table, W1, b1, W2, b2):
    full = lambda shape: pl.BlockSpec(shape, lambda: (0,) * len(shape))
    return pl.pallas_call(
        _table_body,
        in_specs=[
            full((NI, C)),
            full((NJ, C)),
            full((NK, C)),
            full((3 * C, C)),
            full((C, 1)),
            full((C, C)),
            full((C, 1)),
        ],
        out_specs=full((C // 2, NCOMBO)),
        out_shape=jax.ShapeDtypeStruct((C // 2, NCOMBO), jnp.int32),
    )(atom_table, residue_table, pos_table, W1, b1.reshape(C, 1),
      W2, b2.reshape(C, 1))


def _make_gather(B, L):
    info = plsc.get_sparse_core_info()
    nc = info.num_cores
    nbb = B // 128
    qbb = nbb // 4
    nfl = 4
    fbb = qbb // nfl
    qtok = qbb * 128
    mesh = plsc.VectorSubcoreMesh(core_axis_name="c", subcore_axis_name="s")

    @functools.partial(
        pl.kernel,
        out_type=jax.ShapeDtypeStruct((L, C // 8, nbb, 8 * 128), jnp.float32),
        mesh=mesh,
        compiler_params=pltpu.CompilerParams(use_tc_tiling_on_sc=False,
                                             needs_layout_passes=False),
        scratch_types=[
            pltpu.VMEM((4, NCOMBO), jnp.int32),
            pltpu.VMEM((2, qtok), jnp.int32),
            pltpu.VMEM((2, fbb, 8 * 128), jnp.float32),
            pltpu.SemaphoreType.DMA,
            pltpu.SemaphoreType.DMA,
            pltpu.SemaphoreType.DMA,
        ],
    )
    def gather_k(pt_hbm, idx_hbm, out_hbm, pt_v, idx_v, stage_v, sem_i,
                 sem_s0, sem_s1):
        sem_s = (sem_s0, sem_s1)
        wid = lax.axis_index("s") * nc + lax.axis_index("c")
        c8 = wid // 4
        q = wid % 4
        for p in range(4):
            pltpu.sync_copy(pt_hbm.at[c8 * 4 + p], pt_v.at[p])
        pltpu.async_copy(idx_hbm.at[0, pl.ds(q * qtok, qtok)], idx_v.at[0],
                         sem_i)

        def l_body(l, carry):
            lb = lax.rem(l, 2)
            pltpu.make_async_copy(idx_hbm.at[l, pl.ds(q * qtok, qtok)],
                                  idx_v.at[lb], sem_i).wait()

            @pl.when(l + 1 < L)
            def _fire_idx():
                pltpu.async_copy(idx_hbm.at[l + 1, pl.ds(q * qtok, qtok)],
                                 idx_v.at[1 - lb], sem_i)

            for f in range(nfl):
                slot = f % 2

                def _drain(slot=slot):
                    pltpu.make_async_copy(
                        out_hbm.at[0, 0, pl.ds(0, fbb), :],
                        stage_v.at[slot], sem_s[slot],
                    ).wait()

                if f >= 2:
                    _drain()
                else:
                    pl.when(l >= 1)(_drain)

                @plsc.parallel_loop(0, fbb * 8, unroll=8)
                def _groups(g, f=f, slot=slot):
                    vidx = idx_v[lb, pl.ds(f * (fbb * 128) + g * 16, 16)]
                    col = (g % 8) * 16
                    for p in range(4):
                        w = plsc.load_gather(pt_v.at[p], [vidx])
                        bf = plsc.bitcast(w, jnp.bfloat16)
                        v_e, v_o = plsc.unpack(
                            bf, format=plsc.PackFormat.INTERLEAVED,
                            preferred_element_type=jnp.float32)
                        stage_v[slot, g // 8, pl.ds(2 * p * 128 + col, 16)] = v_e
                        stage_v[slot, g // 8, pl.ds((2 * p + 1) * 128 + col, 16)] = v_o

                pltpu.async_copy(
                    stage_v.at[slot],
                    out_hbm.at[l, c8, pl.ds(q * qbb + f * fbb, fbb), :],
                    sem_s[slot],
                )
            return carry

        lax.fori_loop(0, L, l_body, 0)
        for slot in range(2):
            pltpu.make_async_copy(
                out_hbm.at[0, 0, pl.ds(0, fbb), :],
                stage_v.at[slot], sem_s[slot],
            ).wait()

    return gather_k


def kernel(atom_type, residue_type, residue_pos, atom_table, residue_table,
           pos_table, W1, b1, W2, b2):
    B, L = atom_type.shape
    PT = _build_table_packed(atom_table, residue_table, pos_table, W1, b1, W2, b2)
    cidx_t = (atom_type.T * NJK + residue_type.T * NK + residue_pos.T
              ).astype(jnp.int32)
    out5 = _make_gather(B, L)(PT, cidx_t)
    out = out5.reshape(L, 8, B // 128, 8, 128).transpose((2, 4, 0, 1, 3))
    return out.reshape(B, L, C)
